# Initial kernel scaffold; baseline (speedup 1.0000x reference)
import jax
import jax.numpy as jnp
from jax.experimental import pallas as pl


def kernel(x, edge_index, batch, W1l, b1l, W1r, W2l, b2l, W2r, Wc, bc):
    raise NotImplementedError("write your pallas kernel here")


# speedup vs baseline: 12.0382x; 12.0382x over previous
import jax
import jax.numpy as jnp
from jax import lax
from jax.experimental import pallas as pl
from jax.experimental.pallas import tpu as pltpu
from jax.experimental.pallas import tpu_sc as plsc

N = 10000
E = 320000
D = 128
DH = 256
B = 64
NCLS = 16

NCORES = 2
NSUB = 16
NW = NCORES * NSUB
EPW = E // NW
K = 125
G = 8
T = EPW // (G * K)
APAD = 10240

_MESH = plsc.VectorSubcoreMesh(core_axis_name="c", subcore_axis_name="s",
                               num_cores=NCORES, num_subcores=NSUB)


def _drain(acc, o0, o1, c, s):
  def go(out):
    @pl.when(s < 15)
    def _():
      pltpu.sync_copy(acc.at[pl.ds(s * 624, 624)], out.at[pl.ds(s * 624, 624)])

    @pl.when(s == 15)
    def _():
      pltpu.sync_copy(acc.at[pl.ds(9360, 640)], out.at[pl.ds(9360, 640)])

  @pl.when(c == 0)
  def _():
    go(o0)

  @pl.when(c == 1)
  def _():
    go(o1)


def _cnt_body(dstr, cnt0, cnt1, idx_d, ones, zc, cacc, sem):
  c = lax.axis_index("c")
  s = lax.axis_index("s")
  w = c * NSUB + s

  o16 = jnp.ones((16,), jnp.float32)
  z16 = jnp.zeros((16,), jnp.float32)

  @pl.loop(0, K)
  def _(r):
    ones[r] = o16

  @pl.loop(0, 160)
  def _(r):
    zc[r] = z16

  @pl.loop(0, 4)
  def _(t):
    pltpu.sync_copy(zc, cacc.at[pl.ds(s * 640 + t * 160, 160)])

  plsc.subcore_barrier()

  @pl.loop(0, T)
  def _(t):
    pltpu.sync_copy(dstr.at[w, t], idx_d)

    @pl.loop(0, G)
    def _(g):
      pltpu.sync_copy(ones, cacc.at[idx_d.at[g]], add=True)

  plsc.subcore_barrier()
  _drain(cacc, cnt0, cnt1, c, s)


_sc_count = pl.kernel(
    _cnt_body,
    out_type=[jax.ShapeDtypeStruct((N, 16), jnp.float32)] * 2,
    mesh=_MESH,
    scratch_types=[pltpu.VMEM((G, K), jnp.int32),
                   pltpu.VMEM((K, 16), jnp.float32),
                   pltpu.VMEM((160, 16), jnp.float32),
                   pltpu.VMEM_SHARED((APAD, 16), jnp.float32),
                   pltpu.SemaphoreType.DMA],
    name="sc_edge_count")


def _agg_body(table, srcr, dstr, out0, out1, idx_s, idx_d, buf, acc,
              sem_a, sem_b):
  c = lax.axis_index("c")
  s = lax.axis_index("s")
  w = c * NSUB + s

  z16 = jnp.zeros((16,), jnp.float32)

  @pl.loop(0, 120)
  def _(r):
    @pl.loop(0, D // 16)
    def _(q):
      buf[0, r, pl.ds(q * 16, 16)] = z16

  @pl.loop(0, 5)
  def _(t):
    pltpu.sync_copy(buf.at[0, pl.ds(0, 120)],
                    acc.at[pl.ds(s * 640 + t * 120, 120)])
  pltpu.sync_copy(buf.at[0, pl.ds(0, 40)], acc.at[pl.ds(s * 640 + 600, 40)])

  plsc.subcore_barrier()

  @pl.loop(0, T)
  def _(t):
    pltpu.sync_copy(srcr.at[w, t], idx_s)
    pltpu.sync_copy(dstr.at[w, t], idx_d)
    pltpu.async_copy(table.at[idx_s.at[0]], buf.at[0], sem_a)
    for g in range(G):
      d = g % 2
      pltpu.make_async_copy(table.at[idx_s.at[g]], buf.at[d],
                            sem_a if d == 0 else sem_b).wait()
      if g + 1 < G:
        pltpu.async_copy(table.at[idx_s.at[g + 1]], buf.at[1 - d],
                         sem_b if d == 0 else sem_a)
      pltpu.sync_copy(buf.at[d], acc.at[idx_d.at[g]], add=True)

  plsc.subcore_barrier()
  _drain(acc, out0, out1, c, s)


_sc_agg = pl.kernel(
    _agg_body,
    out_type=[jax.ShapeDtypeStruct((N, D), jnp.float32)] * 2,
    mesh=_MESH,
    scratch_types=[pltpu.VMEM((G, K), jnp.int32),
                   pltpu.VMEM((G, K), jnp.int32),
                   pltpu.VMEM((2, K, D), jnp.float32),
                   pltpu.VMEM_SHARED((APAD, D), jnp.float32),
                   pltpu.SemaphoreType.DMA,
                   pltpu.SemaphoreType.DMA],
    name="sc_edge_agg")


_R = 1000


def _dot_t(a, b):
  return lax.dot_general(a, b, (((1,), (1,)), ((), ())),
                         preferred_element_type=jnp.float32)


def _dense1_body(a0, a1, c0, c1, xb, w1l, b1l, w1r, w2l, w2r, b2l,
                 p_out, q_out):
  cnt = jnp.maximum(c0[:, 0:1] + c1[:, 0:1], 1.0)
  aggmean = (a0[...] + a1[...]) / cnt
  h = _dot_t(aggmean, w1l[...]) + b1l[...] + _dot_t(xb[...], w1r[...])
  h = jnp.maximum(h, 0.0)
  p_out[...] = _dot_t(h, w2l[...])
  q_out[...] = _dot_t(h, w2r[...]) + b2l[...]


def _dense1(a0, a1, c0, c1, x, w1l, b1l, w1r, w2l, w2r, b2l):
  row = pl.BlockSpec((_R, D), lambda i: (i, 0))
  cntspec = pl.BlockSpec((_R, 16), lambda i: (i, 0))
  full = lambda shape: pl.BlockSpec(shape, lambda i: (0, 0))
  return pl.pallas_call(
      _dense1_body,
      grid=(N // _R,),
      in_specs=[row, row, cntspec, cntspec, row,
                full((DH, D)), full((1, DH)), full((DH, D)),
                full((D, DH)), full((D, DH)), full((1, D))],
      out_specs=[row, row],
      out_shape=[jax.ShapeDtypeStruct((N, D), jnp.float32)] * 2,
  )(a0, a1, c0, c1, x, w1l, b1l, w1r, w2l, w2r, b2l)


def _dense2_body(a0, a1, c0, c1, qb, bb, wc, bc, out, psum, pcnt):
  i = pl.program_id(0)

  @pl.when(i == 0)
  def _():
    psum[...] = jnp.zeros_like(psum)
    pcnt[...] = jnp.zeros_like(pcnt)

  cnt = jnp.maximum(c0[:, 0:1] + c1[:, 0:1], 1.0)
  h2 = jnp.maximum((a0[...] + a1[...]) / cnt + qb[...], 0.0)
  bv = bb[0]
  rows = lax.broadcasted_iota(jnp.int32, (B, _R), 0)
  oh = (rows == bv).astype(jnp.float32)
  psum[...] += jnp.dot(oh, h2, preferred_element_type=jnp.float32)
  pcnt[...] += jnp.sum(oh, axis=1, keepdims=True)

  @pl.when(i == N // _R - 1)
  def _():
    pooled = psum[...] / jnp.maximum(pcnt[...], 1.0)
    out[...] = _dot_t(pooled, wc[...]) + bc[...]


def _dense2(a0, a1, c0, c1, q, batch3, wc, bc):
  row = pl.BlockSpec((_R, D), lambda i: (i, 0))
  cntspec = pl.BlockSpec((_R, 16), lambda i: (i, 0))
  return pl.pallas_call(
      _dense2_body,
      grid=(N // _R,),
      in_specs=[row, row, cntspec, cntspec, row,
                pl.BlockSpec((1, 1, _R), lambda i: (i, 0, 0)),
                pl.BlockSpec((NCLS, D), lambda i: (0, 0)),
                pl.BlockSpec((1, NCLS), lambda i: (0, 0))],
      out_specs=pl.BlockSpec((B, NCLS), lambda i: (0, 0)),
      out_shape=jax.ShapeDtypeStruct((B, NCLS), jnp.float32),
      scratch_shapes=[pltpu.VMEM((B, D), jnp.float32),
                      pltpu.VMEM((B, 1), jnp.float32)],
  )(a0, a1, c0, c1, q, batch3, wc, bc)


def kernel(x, edge_index, batch, W1l, b1l, W1r, W2l, b2l, W2r, Wc, bc):
  src = edge_index[0].reshape(NW, T, G, K)
  dst = edge_index[1].reshape(NW, T, G, K)

  c0, c1 = _sc_count(dst)
  a0, a1 = _sc_agg(x, src, dst)
  p, q = _dense1(a0, a1, c0, c1, x,
                 W1l, b1l.reshape(1, DH), W1r, W2l, W2r, b2l.reshape(1, D))
  g0, g1 = _sc_agg(p, src, dst)
  return _dense2(g0, g1, c0, c1, q, batch.reshape(N // _R, 1, _R),
                 Wc, bc.reshape(1, NCLS))
